# baseline (speedup 1.0000x reference)
import functools

import jax
import jax.numpy as jnp
from jax import lax
from jax.experimental import pallas as pl
from jax.experimental.pallas import tpu as pltpu
from jax.experimental.pallas import tpu_sc as plsc

N, D_IN, D_OUT, C = 8192, 512, 512, 64
BIG = 2147483647

NC, NS = 2, 16
NW = NC * NS
ROWS_W = N // NW
SCCHUNK = 64
NCHUNK = ROWS_W // SCCHUNK
NV = D_IN // 16

_sc_mesh = plsc.VectorSubcoreMesh(core_axis_name="c", subcore_axis_name="s")


@functools.partial(
    pl.kernel,
    out_type=jax.ShapeDtypeStruct((NW, C, D_IN), jnp.float32),
    mesh=_sc_mesh,
    scratch_types=[
        pltpu.VMEM((2, SCCHUNK, D_IN), jnp.float32),
        pltpu.VMEM((NCHUNK, SCCHUNK), jnp.int32),
        pltpu.VMEM((C, D_IN), jnp.float32),
        pltpu.SemaphoreType.DMA,
        pltpu.SemaphoreType.DMA,
    ],
)
def _sc_segsum(x_hbm, lab_hbm, zeros_hbm, sums_hbm, xbuf, labv, acc,
               sem0, sem1):
    c = lax.axis_index("c")
    s = lax.axis_index("s")
    wid = c * NS + s

    pltpu.sync_copy(zeros_hbm, acc)
    pltpu.sync_copy(lab_hbm.at[wid], labv)

    sems = (sem0, sem1)
    cps = [None, None]
    cps[0] = pltpu.async_copy(x_hbm.at[wid, 0], xbuf.at[0], sems[0])
    for i in range(NCHUNK):
        cps[i % 2].wait()
        if i + 1 < NCHUNK:
            cps[(i + 1) % 2] = pltpu.async_copy(
                x_hbm.at[wid, i + 1], xbuf.at[(i + 1) % 2], sems[(i + 1) % 2])

        def row_group(g, carry, i=i):
            labvec = labv[i, pl.ds(g * 16, 16)]
            for r16 in range(16):
                lab = labvec[r16]
                r = g * 16 + r16
                for jj in range(NV):
                    plsc.addupdate(
                        acc.at[lab, pl.ds(jj * 16, 16)],
                        xbuf[i % 2, r, pl.ds(jj * 16, 16)],
                    )
            return carry

        lax.fori_loop(0, SCCHUNK // 16, row_group, 0)

    pltpu.sync_copy(acc, sums_hbm.at[wid])


def _tc_body(psums_ref, labels_ref, W_ref, b_ref, out_ref):
    sums = psums_ref[0]
    for t in range(1, NW):
        sums = sums + psums_ref[t]

    labels = labels_ref[...]
    lab_b = jnp.broadcast_to(labels, (C, N))
    class_ids = lax.broadcasted_iota(jnp.int32, (C, N), 0)
    onehot = lab_b == class_ids
    cnt_col = jnp.sum(onehot.astype(jnp.float32), axis=1, keepdims=True)
    row_idx = lax.broadcasted_iota(jnp.int32, (C, N), 1)
    fi_col_i = jnp.min(jnp.where(onehot, row_idx, BIG), axis=1, keepdims=True)

    means = sums / jnp.broadcast_to(cnt_col, (C, D_IN))
    proto = lax.dot_general(
        means, W_ref[...], dimension_numbers=(((1,), (0,)), ((), ())),
        preferred_element_type=jnp.float32, precision=lax.Precision.HIGHEST,
    ) + b_ref[...]

    fi_col = fi_col_i.astype(jnp.float32)
    eye = (lax.broadcasted_iota(jnp.int32, (C, C), 0)
           == lax.broadcasted_iota(jnp.int32, (C, C), 1)).astype(jnp.float32)
    fi_rowv = lax.dot_general(
        fi_col, eye, dimension_numbers=(((0,), (0,)), ((), ())),
        preferred_element_type=jnp.float32, precision=lax.Precision.HIGHEST,
    )
    fi_lanes = jnp.broadcast_to(fi_rowv, (C, C))
    fi_subl = jnp.broadcast_to(fi_col, (C, C))
    lane_id = lax.broadcasted_iota(jnp.int32, (C, C), 1)
    subl_id = lax.broadcasted_iota(jnp.int32, (C, C), 0)
    less = (fi_lanes < fi_subl) | ((fi_lanes == fi_subl) & (lane_id < subl_id))
    rank_col = jnp.sum(less.astype(jnp.float32), axis=1, keepdims=True)
    rank_rowv = lax.dot_general(
        rank_col, eye, dimension_numbers=(((0,), (0,)), ((), ())),
        preferred_element_type=jnp.float32, precision=lax.Precision.HIGHEST,
    )
    perm = (jnp.broadcast_to(rank_rowv, (C, C))
            == subl_id.astype(jnp.float32)).astype(jnp.float32)
    out_ref[...] = lax.dot_general(
        perm, proto, dimension_numbers=(((1,), (0,)), ((), ())),
        preferred_element_type=jnp.float32, precision=lax.Precision.HIGHEST,
    )


def _tc_finish(psums, labels2d, W, b2d):
    return pl.pallas_call(
        _tc_body,
        in_specs=[
            pl.BlockSpec((NW, C, D_IN), lambda: (0, 0, 0)),
            pl.BlockSpec((1, N), lambda: (0, 0)),
            pl.BlockSpec((D_IN, D_OUT), lambda: (0, 0)),
            pl.BlockSpec((1, D_OUT), lambda: (0, 0)),
        ],
        out_specs=pl.BlockSpec((C, D_OUT), lambda: (0, 0)),
        out_shape=jax.ShapeDtypeStruct((C, D_OUT), jnp.float32),
    )(psums, labels2d, W, b2d)


@jax.jit
def kernel(inputs, labels, W, b):
    labels_flat = labels.reshape(N)
    x4d = inputs.reshape(NW, NCHUNK, SCCHUNK, D_IN)
    lab3d = labels_flat.reshape(NW, NCHUNK, SCCHUNK)
    zeros = jnp.zeros((C, D_IN), jnp.float32)
    psums = _sc_segsum(x4d, lab3d, zeros)
    return _tc_finish(psums, labels_flat.reshape(1, N), W, b.reshape(1, D_OUT))


# speedup vs baseline: 1.4414x; 1.4414x over previous
import functools

import jax
import jax.numpy as jnp
from jax import lax
from jax.experimental import pallas as pl
from jax.experimental.pallas import tpu as pltpu
from jax.experimental.pallas import tpu_sc as plsc

N, D_IN, D_OUT, C = 8192, 512, 512, 64
BIG = 2147483647

NC, NS = 2, 16
NW = NC * NS
ROWS_W = N // NW
SCCHUNK = 32
NCHUNK = ROWS_W // SCCHUNK
NV = D_IN // 16

_sc_mesh = plsc.VectorSubcoreMesh(core_axis_name="c", subcore_axis_name="s")


@functools.partial(
    pl.kernel,
    out_type=jax.ShapeDtypeStruct((NW, C, D_IN), jnp.float32),
    mesh=_sc_mesh,
    scratch_types=[
        pltpu.VMEM((2, SCCHUNK, D_IN), jnp.float32),
        pltpu.VMEM((NCHUNK, SCCHUNK), jnp.int32),
        pltpu.VMEM((C, D_IN), jnp.float32),
        pltpu.VMEM((C, D_IN), jnp.float32),
        pltpu.SemaphoreType.DMA,
        pltpu.SemaphoreType.DMA,
    ],
)
def _sc_segsum(x_hbm, lab_hbm, sums_hbm, xbuf, labv, acc0, acc1, sem0, sem1):
    c = lax.axis_index("c")
    s = lax.axis_index("s")
    wid = c * NS + s

    pltpu.sync_copy(lab_hbm.at[wid], labv)
    pltpu.async_copy(x_hbm.at[wid, 0], xbuf.at[0], sem0)

    zv = jnp.zeros((16,), jnp.float32)

    def zero_row(r, carry):
        for jj in range(NV):
            acc0[r, pl.ds(jj * 16, 16)] = zv
            acc1[r, pl.ds(jj * 16, 16)] = zv
        return carry

    lax.fori_loop(0, C, zero_row, 0)

    def process(buf, chunk):
        def row_group(g, carry):
            labvec = labv[chunk, pl.ds(g * 16, 16)]
            for r16 in range(16):
                lab = labvec[r16]
                dst = acc0 if r16 % 2 == 0 else acc1
                row = g * 16 + r16
                vals = [xbuf[buf, row, pl.ds(jj * 16, 16)] for jj in range(NV)]
                for jj in range(NV):
                    plsc.addupdate(dst.at[lab, pl.ds(jj * 16, 16)], vals[jj])
            return carry

        lax.fori_loop(0, SCCHUNK // 16, row_group, 0)

    def pair_body(i2, carry):
        ca = 2 * i2
        pltpu.make_async_copy(x_hbm.at[wid, ca], xbuf.at[0], sem0).wait()
        pltpu.async_copy(x_hbm.at[wid, ca + 1], xbuf.at[1], sem1)
        process(0, ca)
        pltpu.make_async_copy(x_hbm.at[wid, ca + 1], xbuf.at[1], sem1).wait()

        @pl.when(i2 + 1 < NCHUNK // 2)
        def _next():
            pltpu.async_copy(x_hbm.at[wid, ca + 2], xbuf.at[0], sem0)

        process(1, ca + 1)
        return carry

    lax.fori_loop(0, NCHUNK // 2, pair_body, 0)

    def merge_row(r, carry):
        for jj in range(NV):
            sl = pl.ds(jj * 16, 16)
            acc0[r, sl] = acc0[r, sl] + acc1[r, sl]
        return carry

    lax.fori_loop(0, C, merge_row, 0)
    pltpu.sync_copy(acc0, sums_hbm.at[wid])


def _tc_body(psums_ref, labels_ref, W_ref, b_ref, out_ref):
    sums = psums_ref[0]
    for t in range(1, NW):
        sums = sums + psums_ref[t]

    labels = labels_ref[...]
    lab_b = jnp.broadcast_to(labels, (C, N))
    class_ids = lax.broadcasted_iota(jnp.int32, (C, N), 0)
    onehot = lab_b == class_ids
    cnt_col = jnp.sum(onehot.astype(jnp.float32), axis=1, keepdims=True)
    row_idx = lax.broadcasted_iota(jnp.int32, (C, N), 1)
    fi_col_i = jnp.min(jnp.where(onehot, row_idx, BIG), axis=1, keepdims=True)

    means = sums / jnp.broadcast_to(cnt_col, (C, D_IN))
    proto = lax.dot_general(
        means, W_ref[...], dimension_numbers=(((1,), (0,)), ((), ())),
        preferred_element_type=jnp.float32, precision=lax.Precision.HIGHEST,
    ) + b_ref[...]

    fi_col = fi_col_i.astype(jnp.float32)
    eye = (lax.broadcasted_iota(jnp.int32, (C, C), 0)
           == lax.broadcasted_iota(jnp.int32, (C, C), 1)).astype(jnp.float32)
    fi_rowv = lax.dot_general(
        fi_col, eye, dimension_numbers=(((0,), (0,)), ((), ())),
        preferred_element_type=jnp.float32, precision=lax.Precision.HIGHEST,
    )
    fi_lanes = jnp.broadcast_to(fi_rowv, (C, C))
    fi_subl = jnp.broadcast_to(fi_col, (C, C))
    lane_id = lax.broadcasted_iota(jnp.int32, (C, C), 1)
    subl_id = lax.broadcasted_iota(jnp.int32, (C, C), 0)
    less = (fi_lanes < fi_subl) | ((fi_lanes == fi_subl) & (lane_id < subl_id))
    rank_col = jnp.sum(less.astype(jnp.float32), axis=1, keepdims=True)
    rank_rowv = lax.dot_general(
        rank_col, eye, dimension_numbers=(((0,), (0,)), ((), ())),
        preferred_element_type=jnp.float32, precision=lax.Precision.HIGHEST,
    )
    perm = (jnp.broadcast_to(rank_rowv, (C, C))
            == subl_id.astype(jnp.float32)).astype(jnp.float32)
    out_ref[...] = lax.dot_general(
        perm, proto, dimension_numbers=(((1,), (0,)), ((), ())),
        preferred_element_type=jnp.float32, precision=lax.Precision.HIGHEST,
    )


def _tc_finish(psums, labels2d, W, b2d):
    return pl.pallas_call(
        _tc_body,
        in_specs=[
            pl.BlockSpec((NW, C, D_IN), lambda: (0, 0, 0)),
            pl.BlockSpec((1, N), lambda: (0, 0)),
            pl.BlockSpec((D_IN, D_OUT), lambda: (0, 0)),
            pl.BlockSpec((1, D_OUT), lambda: (0, 0)),
        ],
        out_specs=pl.BlockSpec((C, D_OUT), lambda: (0, 0)),
        out_shape=jax.ShapeDtypeStruct((C, D_OUT), jnp.float32),
    )(psums, labels2d, W, b2d)


@jax.jit
def kernel(inputs, labels, W, b):
    labels_flat = labels.reshape(N)
    x4d = inputs.reshape(NW, NCHUNK, SCCHUNK, D_IN)
    lab3d = labels_flat.reshape(NW, NCHUNK, SCCHUNK)
    psums = _sc_segsum(x4d, lab3d)
    return _tc_finish(psums, labels_flat.reshape(1, N), W, b.reshape(1, D_OUT))
